# baseline (speedup 1.0000x reference)
import functools

import jax
import jax.numpy as jnp
from jax import lax
from jax.experimental import pallas as pl
from jax.experimental.pallas import tpu as pltpu
from jax.experimental.pallas import tpu_sc as plsc

NV = 50000
N_LIT = 2 * NV
N_CLAUSES = 200000
E = 1600000
INV_T = float(2.0 ** 1.2)

NCHUNK = E // 128
NW = 32
BASE_CH = NCHUNK // NW
EXTRA = NCHUNK - BASE_CH * NW
GE = 64
GEL = GE * 128
FULL_GROUPS = BASE_CH // GE
KK = FULL_GROUPS // 2

ACC = 200192
ACC_SL = ACC // 16
ZHALF = ACC_SL // 2
EMB_SL = 3120
EMB_CP = 3200
M_EDGE = (16 * BASE_CH + min(16, EXTRA)) * 128


def _sc_body(emb_h, src_h, dst_h, out_s,
             lit_sp, num_sp, den_sp,
             embw, negw, zbuf,
             src_b0, src_b1, dst_b0, dst_b1,
             v_b0, v_b1, a_b0, a_b1, b_b0, b_b1,
             srct, dstt, v_row, a_row, b_row,
             mew, cbw, nbrow, dbrow, s1acc, s2acc,
             gsem0, gsem1, ssem0, ssem1,
             srcsem0, srcsem1, dstsem0, dstsem1, zsem, esem):
    c = lax.axis_index("c")
    s = lax.axis_index("s")
    wid = c * 16 + s

    cb = wid * BASE_CH + jnp.minimum(wid, EXTRA)
    nch = jnp.where(wid < EXTRA, BASE_CH + 1, BASE_CH)
    eb = cb * 128

    def src_sl(g):
        return src_h.at[pl.ds(eb + g * GEL, GEL)]

    def dst_sl(g):
        return dst_h.at[pl.ds(eb + g * GEL, GEL)]

    pltpu.async_copy(src_sl(0), src_b0, srcsem0)
    pltpu.async_copy(src_sl(1), src_b1, srcsem1)
    pltpu.async_copy(dst_sl(0), dst_b0, dstsem0)
    pltpu.async_copy(emb_h.at[pl.ds(s * EMB_SL, EMB_CP)], embw, esem)

    def z16(i, _):
        zbuf[pl.ds(i * 16, 16)] = jnp.zeros((16,), jnp.float32)
        return _
    lax.fori_loop(0, ZHALF // 16, z16, None)
    zb = s * ACC_SL
    pltpu.async_copy(zbuf, num_sp.at[pl.ds(zb, ZHALF)], zsem)
    pltpu.async_copy(zbuf, num_sp.at[pl.ds(zb + ZHALF, ZHALF)], zsem)
    pltpu.async_copy(zbuf, den_sp.at[pl.ds(zb, ZHALF)], zsem)
    pltpu.async_copy(zbuf, den_sp.at[pl.ds(zb + ZHALF, ZHALF)], zsem)

    off = s * EMB_SL
    pltpu.make_async_copy(emb_h.at[pl.ds(off, EMB_CP)], embw, esem).wait()

    def neg(i, _):
        negw[pl.ds(i * 16, 16)] = 1.0 - embw[pl.ds(i * 16, 16)]
        return _
    lax.fori_loop(0, EMB_CP // 16, neg, None)
    pltpu.async_copy(embw, lit_sp.at[pl.ds(off, EMB_CP)], zsem)
    pltpu.async_copy(negw, lit_sp.at[pl.ds(NV + off, EMB_CP)], zsem)
    for _ in range(4):
        pltpu.make_async_copy(zbuf, num_sp.at[pl.ds(zb, ZHALF)], zsem).wait()
    pltpu.make_async_copy(embw, lit_sp.at[pl.ds(off, EMB_CP)], zsem).wait()
    pltpu.make_async_copy(negw, lit_sp.at[pl.ds(NV + off, EMB_CP)], zsem).wait()
    plsc.subcore_barrier()

    def compute(v_b, a_b, b_b):
        def cstep(i, _):
            for u in range(4):
                sl = pl.ds(i * 64 + u * 16, 16)
                v = v_b[sl]
                e = jnp.exp(v * INV_T)
                a_b[sl] = v * e
                b_b[sl] = e
            return _
        lax.fori_loop(0, GEL // 64, cstep, None)

    pltpu.make_async_copy(src_sl(0), src_b0, srcsem0).wait()
    pltpu.async_copy(lit_sp.at[src_b0], v_b0, gsem0)

    def body(k, _):
        g0 = k * 2
        pltpu.make_async_copy(lit_sp.at[src_b0], v_b0, gsem0).wait()

        @pl.when(k < KK - 1)
        def _():
            pltpu.async_copy(src_sl(g0 + 2), src_b0, srcsem0)
        pltpu.make_async_copy(src_sl(g0 + 1), src_b1, srcsem1).wait()
        pltpu.async_copy(lit_sp.at[src_b1], v_b1, gsem1)
        compute(v_b0, a_b0, b_b0)
        pltpu.make_async_copy(dst_sl(g0), dst_b0, dstsem0).wait()

        @pl.when(k > 0)
        def _():
            pltpu.make_async_copy(a_b1, num_sp.at[dst_b1], ssem1).wait()
            pltpu.make_async_copy(b_b1, den_sp.at[dst_b1], ssem1).wait()
        pltpu.async_copy(dst_sl(g0 + 1), dst_b1, dstsem1)
        pltpu.async_copy(a_b0, num_sp.at[dst_b0], ssem0, add=True)
        pltpu.async_copy(b_b0, den_sp.at[dst_b0], ssem0, add=True)

        pltpu.make_async_copy(lit_sp.at[src_b1], v_b1, gsem1).wait()

        @pl.when(k < KK - 1)
        def _():
            pltpu.async_copy(src_sl(g0 + 3), src_b1, srcsem1)
            pltpu.make_async_copy(src_sl(g0 + 2), src_b0, srcsem0).wait()
            pltpu.async_copy(lit_sp.at[src_b0], v_b0, gsem0)
        compute(v_b1, a_b1, b_b1)
        pltpu.make_async_copy(dst_sl(g0 + 1), dst_b1, dstsem1).wait()
        pltpu.make_async_copy(a_b0, num_sp.at[dst_b0], ssem0).wait()
        pltpu.make_async_copy(b_b0, den_sp.at[dst_b0], ssem0).wait()

        @pl.when(k < KK - 1)
        def _():
            pltpu.async_copy(dst_sl(g0 + 2), dst_b0, dstsem0)
        pltpu.async_copy(a_b1, num_sp.at[dst_b1], ssem1, add=True)
        pltpu.async_copy(b_b1, den_sp.at[dst_b1], ssem1, add=True)
        return _
    lax.fori_loop(0, KK, body, None)
    pltpu.make_async_copy(a_b1, num_sp.at[dst_b1], ssem1).wait()
    pltpu.make_async_copy(b_b1, den_sp.at[dst_b1], ssem1).wait()

    def tail(j, _):
        rb = eb + FULL_GROUPS * GEL + j * 128
        pltpu.sync_copy(src_h.at[pl.ds(rb, 128)], srct)
        pltpu.sync_copy(dst_h.at[pl.ds(rb, 128)], dstt)
        pltpu.sync_copy(lit_sp.at[srct], v_row)
        for i in range(8):
            sl = pl.ds(i * 16, 16)
            v = v_row[sl]
            e = jnp.exp(v * INV_T)
            a_row[sl] = v * e
            b_row[sl] = e
        pltpu.sync_copy(a_row, num_sp.at[dstt], add=True)
        pltpu.sync_copy(b_row, den_sp.at[dstt], add=True)
        return _
    lax.fori_loop(0, nch - FULL_GROUPS * GE, tail, None)
    plsc.subcore_barrier()

    pltpu.sync_copy(dst_h.at[pl.ds(M_EDGE, 16)], cbw)
    cbv = lax.gather(
        cbw[...], jnp.zeros((16, 1), jnp.int32),
        lax.GatherDimensionNumbers(offset_dims=(), collapsed_slice_dims=(0,),
                                   start_index_map=(0,)),
        (1,), mode=lax.GatherScatterMode.PROMISE_IN_BOUNDS)
    mew[...] = cbv
    pltpu.sync_copy(num_sp.at[mew], nbrow)
    pltpu.sync_copy(den_sp.at[mew], dbrow)
    i16f = lax.iota(jnp.int32, 16)
    s1acc[...] = jnp.zeros((16,), jnp.float32)
    s2acc[...] = jnp.zeros((16,), jnp.float32)
    rbase = s * ACC_SL
    for h in range(2):
        pltpu.sync_copy(num_sp.at[pl.ds(rbase + h * ZHALF, ZHALF)], a_b0.at[pl.ds(0, ZHALF)])
        pltpu.sync_copy(den_sp.at[pl.ds(rbase + h * ZHALF, ZHALF)], b_b0.at[pl.ds(0, ZHALF)])

        def red(i, _):
            sl = pl.ds(i * 16, 16)
            rows = rbase + h * ZHALF + i * 16 + i16f
            nm = a_b0[sl]
            dn = b_b0[sl]
            sgn = 1 - 2 * c
            side = rows * sgn < cbv * sgn
            own = (rows < N_CLAUSES) & (rows != cbv) & side
            cval = jnp.where(own & (dn > 0), nm / jnp.maximum(dn, 1e-30), 0.0)
            w2 = jnp.where(own, jnp.exp(-cval * INV_T), 0.0)
            s1acc[...] = s1acc[...] + cval * w2
            s2acc[...] = s2acc[...] + w2
            return _
        lax.fori_loop(0, ZHALF // 16, red, None)
    pltpu.sync_copy(s1acc, out_s.at[c, pl.ds(s * 16, 16)])
    pltpu.sync_copy(s2acc, out_s.at[c, pl.ds(256 + s * 16, 16)])

    @pl.when(s == 0)
    def _():
        pltpu.sync_copy(nbrow, out_s.at[c, pl.ds(512, 16)])
        pltpu.sync_copy(dbrow, out_s.at[c, pl.ds(512 + 16, 16)])


_sc_edge_pass = functools.partial(
    pl.kernel,
    out_type=jax.ShapeDtypeStruct((2, 768), jnp.float32),
    mesh=plsc.VectorSubcoreMesh(core_axis_name="c", subcore_axis_name="s"),
    compiler_params=pltpu.CompilerParams(use_tc_tiling_on_sc=False),
    scratch_types=[
        pltpu.VMEM_SHARED((N_LIT,), jnp.float32),
        pltpu.VMEM_SHARED((ACC,), jnp.float32),
        pltpu.VMEM_SHARED((ACC,), jnp.float32),
        pltpu.VMEM((EMB_CP,), jnp.float32),
        pltpu.VMEM((EMB_CP,), jnp.float32),
        pltpu.VMEM((ZHALF,), jnp.float32),
        pltpu.VMEM((GEL,), jnp.int32),
        pltpu.VMEM((GEL,), jnp.int32),
        pltpu.VMEM((GEL,), jnp.int32),
        pltpu.VMEM((GEL,), jnp.int32),
        pltpu.VMEM((GEL,), jnp.float32),
        pltpu.VMEM((GEL,), jnp.float32),
        pltpu.VMEM((GEL,), jnp.float32),
        pltpu.VMEM((GEL,), jnp.float32),
        pltpu.VMEM((GEL,), jnp.float32),
        pltpu.VMEM((GEL,), jnp.float32),
        pltpu.VMEM((128,), jnp.int32),
        pltpu.VMEM((128,), jnp.int32),
        pltpu.VMEM((128,), jnp.float32),
        pltpu.VMEM((128,), jnp.float32),
        pltpu.VMEM((128,), jnp.float32),
        pltpu.VMEM((16,), jnp.int32),
        pltpu.VMEM((16,), jnp.int32),
        pltpu.VMEM((16,), jnp.float32),
        pltpu.VMEM((16,), jnp.float32),
        pltpu.VMEM((16,), jnp.float32),
        pltpu.VMEM((16,), jnp.float32),
        pltpu.SemaphoreType.DMA,
        pltpu.SemaphoreType.DMA,
        pltpu.SemaphoreType.DMA,
        pltpu.SemaphoreType.DMA,
        pltpu.SemaphoreType.DMA,
        pltpu.SemaphoreType.DMA,
        pltpu.SemaphoreType.DMA,
        pltpu.SemaphoreType.DMA,
        pltpu.SemaphoreType.DMA,
        pltpu.SemaphoreType.DMA,
    ],
)(_sc_body)


def _fin_body(s_ref, o_ref):
    sm = s_ref[...]
    s1 = jnp.sum(sm[0]) + jnp.sum(sm[3])
    s2 = jnp.sum(sm[1]) + jnp.sum(sm[4])
    nb = sm[2, 0] + sm[5, 0]
    db = sm[2, 16] + sm[5, 16]
    cb = jnp.where(db > 0, nb / jnp.maximum(db, 1e-30), 0.0)
    wb = jnp.exp(-cb * INV_T)
    o_ref[...] = ((s1 + cb * wb) / (s2 + wb)).reshape(1, 1)


_finalize = pl.pallas_call(
    _fin_body,
    out_shape=jax.ShapeDtypeStruct((1, 1), jnp.float32),
)


def kernel(emb, or_src, or_dst, epoch):
    del epoch
    sums = _sc_edge_pass(emb, or_src, or_dst)
    res = _finalize(sums.reshape(6, 256))
    return res[0, 0]


# speedup vs baseline: 1.1625x; 1.1625x over previous
import functools

import jax
import jax.numpy as jnp
from jax import lax
from jax.experimental import pallas as pl
from jax.experimental.pallas import tpu as pltpu
from jax.experimental.pallas import tpu_sc as plsc

NV = 50000
N_LIT = 2 * NV
N_CLAUSES = 200000
E = 1600000
INV_T = float(2.0 ** 1.2)

NCHUNK = E // 128
NW = 32
BASE_CH = NCHUNK // NW
EXTRA = NCHUNK - BASE_CH * NW
GE = 64
GEL = GE * 128
FULL_GROUPS = BASE_CH // GE
KK = FULL_GROUPS // 2

ACC = 200192
ACC_SL = ACC // 16
ZHALF = ACC_SL // 2
EMB_SL = 3120
EMB_CP = 3200


def _sc_body(emb_h, src_h, dst_h, out_h,
             lit_sp, num_sp, den_sp,
             embw, negw, zbuf,
             src_b0, src_b1, dst_b0, dst_b1,
             v_b0, v_b1, a_b0, a_b1, b_b0, b_b1,
             srct, dstt, v_row, a_row, b_row,
             gsem0, gsem1, ssem0, ssem1,
             srcsem0, srcsem1, dstsem0, dstsem1, zsem, esem):
    c = lax.axis_index("c")
    s = lax.axis_index("s")
    wid = c * 16 + s

    cb = wid * BASE_CH + jnp.minimum(wid, EXTRA)
    nch = jnp.where(wid < EXTRA, BASE_CH + 1, BASE_CH)
    eb = cb * 128

    def src_sl(g):
        return src_h.at[pl.ds(eb + g * GEL, GEL)]

    def dst_sl(g):
        return dst_h.at[pl.ds(eb + g * GEL, GEL)]

    pltpu.async_copy(src_sl(0), src_b0, srcsem0)
    pltpu.async_copy(src_sl(1), src_b1, srcsem1)
    pltpu.async_copy(dst_sl(0), dst_b0, dstsem0)
    pltpu.async_copy(emb_h.at[pl.ds(s * EMB_SL, EMB_CP)], embw, esem)

    def z16(i, _):
        zbuf[pl.ds(i * 16, 16)] = jnp.zeros((16,), jnp.float32)
        return _
    lax.fori_loop(0, ZHALF // 16, z16, None)
    zb = s * ACC_SL
    pltpu.async_copy(zbuf, num_sp.at[pl.ds(zb, ZHALF)], zsem)
    pltpu.async_copy(zbuf, num_sp.at[pl.ds(zb + ZHALF, ZHALF)], zsem)
    pltpu.async_copy(zbuf, den_sp.at[pl.ds(zb, ZHALF)], zsem)
    pltpu.async_copy(zbuf, den_sp.at[pl.ds(zb + ZHALF, ZHALF)], zsem)

    off = s * EMB_SL
    pltpu.make_async_copy(emb_h.at[pl.ds(off, EMB_CP)], embw, esem).wait()

    def neg(i, _):
        negw[pl.ds(i * 16, 16)] = 1.0 - embw[pl.ds(i * 16, 16)]
        return _
    lax.fori_loop(0, EMB_CP // 16, neg, None)
    pltpu.async_copy(embw, lit_sp.at[pl.ds(off, EMB_CP)], zsem)
    pltpu.async_copy(negw, lit_sp.at[pl.ds(NV + off, EMB_CP)], zsem)
    for _ in range(4):
        pltpu.make_async_copy(zbuf, num_sp.at[pl.ds(zb, ZHALF)], zsem).wait()
    pltpu.make_async_copy(embw, lit_sp.at[pl.ds(off, EMB_CP)], zsem).wait()
    pltpu.make_async_copy(negw, lit_sp.at[pl.ds(NV + off, EMB_CP)], zsem).wait()
    plsc.subcore_barrier()

    def compute(v_b, a_b, b_b):
        def cstep(i, _):
            for u in range(4):
                sl = pl.ds(i * 64 + u * 16, 16)
                v = v_b[sl]
                e = jnp.exp(v * INV_T)
                a_b[sl] = v * e
                b_b[sl] = e
            return _
        lax.fori_loop(0, GEL // 64, cstep, None)

    pltpu.make_async_copy(src_sl(0), src_b0, srcsem0).wait()
    pltpu.async_copy(lit_sp.at[src_b0], v_b0, gsem0)

    def body(k, _):
        g0 = k * 2
        pltpu.make_async_copy(lit_sp.at[src_b0], v_b0, gsem0).wait()

        @pl.when(k < KK - 1)
        def _():
            pltpu.async_copy(src_sl(g0 + 2), src_b0, srcsem0)
        pltpu.make_async_copy(src_sl(g0 + 1), src_b1, srcsem1).wait()
        pltpu.async_copy(lit_sp.at[src_b1], v_b1, gsem1)
        compute(v_b0, a_b0, b_b0)
        pltpu.make_async_copy(dst_sl(g0), dst_b0, dstsem0).wait()

        @pl.when(k > 0)
        def _():
            pltpu.make_async_copy(a_b1, num_sp.at[dst_b1], ssem1).wait()
            pltpu.make_async_copy(b_b1, den_sp.at[dst_b1], ssem1).wait()
        pltpu.async_copy(dst_sl(g0 + 1), dst_b1, dstsem1)
        pltpu.async_copy(a_b0, num_sp.at[dst_b0], ssem0, add=True)
        pltpu.async_copy(b_b0, den_sp.at[dst_b0], ssem0, add=True)

        pltpu.make_async_copy(lit_sp.at[src_b1], v_b1, gsem1).wait()

        @pl.when(k < KK - 1)
        def _():
            pltpu.async_copy(src_sl(g0 + 3), src_b1, srcsem1)
            pltpu.make_async_copy(src_sl(g0 + 2), src_b0, srcsem0).wait()
            pltpu.async_copy(lit_sp.at[src_b0], v_b0, gsem0)
        compute(v_b1, a_b1, b_b1)
        pltpu.make_async_copy(dst_sl(g0 + 1), dst_b1, dstsem1).wait()
        pltpu.make_async_copy(a_b0, num_sp.at[dst_b0], ssem0).wait()
        pltpu.make_async_copy(b_b0, den_sp.at[dst_b0], ssem0).wait()

        @pl.when(k < KK - 1)
        def _():
            pltpu.async_copy(dst_sl(g0 + 2), dst_b0, dstsem0)
        pltpu.async_copy(a_b1, num_sp.at[dst_b1], ssem1, add=True)
        pltpu.async_copy(b_b1, den_sp.at[dst_b1], ssem1, add=True)
        return _
    lax.fori_loop(0, KK, body, None)
    pltpu.make_async_copy(a_b1, num_sp.at[dst_b1], ssem1).wait()
    pltpu.make_async_copy(b_b1, den_sp.at[dst_b1], ssem1).wait()

    def tail(j, _):
        rb = eb + FULL_GROUPS * GEL + j * 128
        pltpu.sync_copy(src_h.at[pl.ds(rb, 128)], srct)
        pltpu.sync_copy(dst_h.at[pl.ds(rb, 128)], dstt)
        pltpu.sync_copy(lit_sp.at[srct], v_row)
        for i in range(8):
            sl = pl.ds(i * 16, 16)
            v = v_row[sl]
            e = jnp.exp(v * INV_T)
            a_row[sl] = v * e
            b_row[sl] = e
        pltpu.sync_copy(a_row, num_sp.at[dstt], add=True)
        pltpu.sync_copy(b_row, den_sp.at[dstt], add=True)
        return _
    lax.fori_loop(0, nch - FULL_GROUPS * GE, tail, None)
    plsc.subcore_barrier()

    wb = s * ACC_SL
    pltpu.sync_copy(num_sp.at[pl.ds(wb, ACC_SL)], out_h.at[c, 0, pl.ds(wb, ACC_SL)])
    pltpu.sync_copy(den_sp.at[pl.ds(wb, ACC_SL)], out_h.at[c, 1, pl.ds(wb, ACC_SL)])


_sc_edge_pass = functools.partial(
    pl.kernel,
    out_type=jax.ShapeDtypeStruct((2, 2, ACC), jnp.float32),
    mesh=plsc.VectorSubcoreMesh(core_axis_name="c", subcore_axis_name="s"),
    compiler_params=pltpu.CompilerParams(use_tc_tiling_on_sc=False),
    scratch_types=[
        pltpu.VMEM_SHARED((N_LIT,), jnp.float32),
        pltpu.VMEM_SHARED((ACC,), jnp.float32),
        pltpu.VMEM_SHARED((ACC,), jnp.float32),
        pltpu.VMEM((EMB_CP,), jnp.float32),
        pltpu.VMEM((EMB_CP,), jnp.float32),
        pltpu.VMEM((ZHALF,), jnp.float32),
        pltpu.VMEM((GEL,), jnp.int32),
        pltpu.VMEM((GEL,), jnp.int32),
        pltpu.VMEM((GEL,), jnp.int32),
        pltpu.VMEM((GEL,), jnp.int32),
        pltpu.VMEM((GEL,), jnp.float32),
        pltpu.VMEM((GEL,), jnp.float32),
        pltpu.VMEM((GEL,), jnp.float32),
        pltpu.VMEM((GEL,), jnp.float32),
        pltpu.VMEM((GEL,), jnp.float32),
        pltpu.VMEM((GEL,), jnp.float32),
        pltpu.VMEM((128,), jnp.int32),
        pltpu.VMEM((128,), jnp.int32),
        pltpu.VMEM((128,), jnp.float32),
        pltpu.VMEM((128,), jnp.float32),
        pltpu.VMEM((128,), jnp.float32),
        pltpu.SemaphoreType.DMA,
        pltpu.SemaphoreType.DMA,
        pltpu.SemaphoreType.DMA,
        pltpu.SemaphoreType.DMA,
        pltpu.SemaphoreType.DMA,
        pltpu.SemaphoreType.DMA,
        pltpu.SemaphoreType.DMA,
        pltpu.SemaphoreType.DMA,
        pltpu.SemaphoreType.DMA,
        pltpu.SemaphoreType.DMA,
    ],
)(_sc_body)


_FIN_ROWS = ACC // 128


def _fin_body(p_ref, o_ref):
    num = p_ref[0, 0] + p_ref[1, 0]
    den = p_ref[0, 1] + p_ref[1, 1]
    row = lax.broadcasted_iota(jnp.int32, (_FIN_ROWS, 128), 0)
    col = lax.broadcasted_iota(jnp.int32, (_FIN_ROWS, 128), 1)
    idx = row * 128 + col
    valid = idx < N_CLAUSES
    cval = jnp.where(valid & (den > 0), num / jnp.maximum(den, 1e-30), 0.0)
    w2 = jnp.where(valid, jnp.exp(-cval * INV_T), 0.0)
    o_ref[...] = (jnp.sum(cval * w2) / jnp.sum(w2)).reshape(1, 1)


_finalize = pl.pallas_call(
    _fin_body,
    out_shape=jax.ShapeDtypeStruct((1, 1), jnp.float32),
)


def kernel(emb, or_src, or_dst, epoch):
    del epoch
    part = _sc_edge_pass(emb, or_src, or_dst)
    res = _finalize(part.reshape(2, 2, _FIN_ROWS, 128))
    return res[0, 0]
